# baseline (speedup 1.0000x reference)
import jax
import jax.numpy as jnp
from jax import lax
from jax.experimental import pallas as pl
from jax.experimental.pallas import tpu as pltpu
from jax.experimental.pallas import tpu_sc as plsc

B = 16384
D = 64
NC = 2
NS = 16
NW = NC * NS
BPW = B // NW
C = 128


def _sc_body(adj4, adj5, cv0, cv1, cv2, cv3, out,
             i4_v, i5_v, r4_v, r5_v, sem0, sem3, sem4, sem5):
    wid = lax.axis_index("s") * NC + lax.axis_index("c")
    base = wid * BPW

    cp0 = pltpu.async_copy(
        cv0.at[pl.ds(base, BPW)],
        out.at[pl.ds(base, BPW), pl.ds(D, D)], sem0)
    cp3 = pltpu.async_copy(
        cv3.at[pl.ds(base, BPW)],
        out.at[pl.ds(base, BPW), pl.ds(3 * D, D)], sem3)

    pltpu.sync_copy(adj5.at[pl.ds(base, BPW)], i5_v)
    pltpu.sync_copy(adj4.at[pl.ds(base, BPW)], i4_v)

    @pl.loop(0, BPW, step=C)
    def _(c):
        g5 = pltpu.async_copy(cv2.at[i5_v.at[pl.ds(c, C)]], r5_v, sem5)
        g4 = pltpu.async_copy(cv1.at[i4_v.at[pl.ds(c, C)]], r4_v, sem4)
        g5.wait()
        pltpu.sync_copy(r5_v, out.at[pl.ds(base + c, C), pl.ds(0, D)])
        g4.wait()
        pltpu.sync_copy(r4_v, out.at[pl.ds(base + c, C), pl.ds(2 * D, D)])

    cp0.wait()
    cp3.wait()


def kernel(adj_list_4, adj_list_5, concat_vecs_0, concat_vecs_1,
           concat_vecs_2, concat_vecs_3):
    i4 = adj_list_4.astype(jnp.int32)
    i5 = adj_list_5.astype(jnp.int32)
    mesh = plsc.VectorSubcoreMesh(core_axis_name="c", subcore_axis_name="s")
    k = pl.kernel(
        _sc_body,
        out_type=jax.ShapeDtypeStruct((B, 4 * D), jnp.float32),
        mesh=mesh,
        compiler_params=pltpu.CompilerParams(use_tc_tiling_on_sc=False),
        scratch_types=[
            pltpu.VMEM((BPW,), jnp.int32),
            pltpu.VMEM((BPW,), jnp.int32),
            pltpu.VMEM((C, D), jnp.float32),
            pltpu.VMEM((C, D), jnp.float32),
            pltpu.SemaphoreType.DMA,
            pltpu.SemaphoreType.DMA,
            pltpu.SemaphoreType.DMA,
            pltpu.SemaphoreType.DMA,
        ],
    )
    return k(i4, i5, concat_vecs_0, concat_vecs_1, concat_vecs_2,
             concat_vecs_3)


# speedup vs baseline: 1.1870x; 1.1870x over previous
import jax
import jax.numpy as jnp
from jax import lax
from jax.experimental import pallas as pl
from jax.experimental.pallas import tpu as pltpu
from jax.experimental.pallas import tpu_sc as plsc

B = 16384
D = 64
V = 1000000
NC = 2
NS = 16
NW = NC * NS
BPW = B // NW
C = 128

RB = 512


def _sc_gather_body(ic4, ic5, t1, t2, ru2_out, ri2_out,
                    i4_v, i5_v, r4_v, r5_v, sem4, sem5):
    wid = lax.axis_index("s") * NC + lax.axis_index("c")
    base = wid * BPW

    pltpu.sync_copy(ic5.at[pl.ds(base, BPW)], i5_v)
    pltpu.sync_copy(ic4.at[pl.ds(base, BPW)], i4_v)

    @pl.loop(0, BPW, step=C)
    def _(c):
        g5 = pltpu.async_copy(t2.at[i5_v.at[pl.ds(c, C)]], r5_v, sem5)
        g4 = pltpu.async_copy(t1.at[i4_v.at[pl.ds(c, C)]], r4_v, sem4)
        g5.wait()
        pltpu.sync_copy(r5_v, ri2_out.at[pl.ds(base + c, C)])
        g4.wait()
        pltpu.sync_copy(r4_v, ru2_out.at[pl.ds(base + c, C)])


def _tc_concat_body(ri2_ref, h5_ref, cv0_ref, ru2_ref, h4_ref, cv3_ref,
                    out_ref):
    sel5 = h5_ref[...] == 1
    out_ref[:, 0 * D:1 * D] = jnp.where(
        sel5, ri2_ref[:, D:2 * D], ri2_ref[:, 0:D])
    out_ref[:, 1 * D:2 * D] = cv0_ref[...]
    sel4 = h4_ref[...] == 1
    out_ref[:, 2 * D:3 * D] = jnp.where(
        sel4, ru2_ref[:, D:2 * D], ru2_ref[:, 0:D])
    out_ref[:, 3 * D:4 * D] = cv3_ref[...]


def kernel(adj_list_4, adj_list_5, concat_vecs_0, concat_vecs_1,
           concat_vecs_2, concat_vecs_3):
    i4 = adj_list_4.astype(jnp.int32)
    i5 = adj_list_5.astype(jnp.int32)
    ic4, h4 = i4 >> 1, i4 & 1
    ic5, h5 = i5 >> 1, i5 & 1
    t1 = concat_vecs_1.reshape(V // 2, 2 * D)
    t2 = concat_vecs_2.reshape(V // 2, 2 * D)

    mesh = plsc.VectorSubcoreMesh(core_axis_name="c", subcore_axis_name="s")
    gather_k = pl.kernel(
        _sc_gather_body,
        out_type=(jax.ShapeDtypeStruct((B, 2 * D), jnp.float32),
                  jax.ShapeDtypeStruct((B, 2 * D), jnp.float32)),
        mesh=mesh,
        scratch_types=[
            pltpu.VMEM((BPW,), jnp.int32),
            pltpu.VMEM((BPW,), jnp.int32),
            pltpu.VMEM((C, 2 * D), jnp.float32),
            pltpu.VMEM((C, 2 * D), jnp.float32),
            pltpu.SemaphoreType.DMA,
            pltpu.SemaphoreType.DMA,
        ],
    )
    ru2, ri2 = gather_k(ic4, ic5, t1, t2)

    spec64 = pl.BlockSpec((RB, D), lambda i: (i, 0))
    spec128 = pl.BlockSpec((RB, 2 * D), lambda i: (i, 0))
    spec1 = pl.BlockSpec((RB, 1), lambda i: (i, 0))
    out = pl.pallas_call(
        _tc_concat_body,
        grid=(B // RB,),
        in_specs=[spec128, spec1, spec64, spec128, spec1, spec64],
        out_specs=pl.BlockSpec((RB, 4 * D), lambda i: (i, 0)),
        out_shape=jax.ShapeDtypeStruct((B, 4 * D), jnp.float32),
    )(ri2, h5[:, None], concat_vecs_0, ru2, h4[:, None], concat_vecs_3)
    return out


# speedup vs baseline: 1.2806x; 1.0788x over previous
import jax
import jax.numpy as jnp
from jax import lax
from jax.experimental import pallas as pl
from jax.experimental.pallas import tpu as pltpu
from jax.experimental.pallas import tpu_sc as plsc

B = 16384
D = 64
V = 1000000
NC = 2
NS = 16
NW = NC * NS
BPW = B // NW
C = 128

RB = 512


def _sc_gather_body(i4h, i5h, t1, t2, ru2_out, ri2_out,
                    i4_v, i5_v, r4_v, r5_v, sem4, sem5):
    wid = lax.axis_index("s") * NC + lax.axis_index("c")
    base = wid * BPW

    pltpu.sync_copy(i5h.at[pl.ds(base, BPW)], i5_v)
    pltpu.sync_copy(i4h.at[pl.ds(base, BPW)], i4_v)

    @pl.loop(0, BPW, step=C)
    def _(c):
        g5 = pltpu.async_copy(t2.at[i5_v.at[pl.ds(c, C)]], r5_v, sem5)
        g4 = pltpu.async_copy(t1.at[i4_v.at[pl.ds(c, C)]], r4_v, sem4)
        g5.wait()
        pltpu.sync_copy(r5_v, ri2_out.at[pl.ds(base + c, C)])
        g4.wait()
        pltpu.sync_copy(r4_v, ru2_out.at[pl.ds(base + c, C)])


def _tc_concat_body(ri2_ref, cv0_ref, ru2_ref, cv3_ref, out_ref):
    out_ref[:, 0 * D:1 * D] = ri2_ref[:, 0:D]
    out_ref[:, 1 * D:2 * D] = cv0_ref[...]
    out_ref[:, 2 * D:3 * D] = ru2_ref[:, 0:D]
    out_ref[:, 3 * D:4 * D] = cv3_ref[...]


def kernel(adj_list_4, adj_list_5, concat_vecs_0, concat_vecs_1,
           concat_vecs_2, concat_vecs_3):
    i4 = adj_list_4.astype(jnp.int32)
    i5 = adj_list_5.astype(jnp.int32)
    t1 = jnp.pad(concat_vecs_1, ((0, 0), (0, D)))
    t2 = jnp.pad(concat_vecs_2, ((0, 0), (0, D)))

    mesh = plsc.VectorSubcoreMesh(core_axis_name="c", subcore_axis_name="s")
    gather_k = pl.kernel(
        _sc_gather_body,
        out_type=(jax.ShapeDtypeStruct((B, 2 * D), jnp.float32),
                  jax.ShapeDtypeStruct((B, 2 * D), jnp.float32)),
        mesh=mesh,
        scratch_types=[
            pltpu.VMEM((BPW,), jnp.int32),
            pltpu.VMEM((BPW,), jnp.int32),
            pltpu.VMEM((C, 2 * D), jnp.float32),
            pltpu.VMEM((C, 2 * D), jnp.float32),
            pltpu.SemaphoreType.DMA,
            pltpu.SemaphoreType.DMA,
        ],
    )
    ru2, ri2 = gather_k(i4, i5, t1, t2)

    spec64 = pl.BlockSpec((RB, D), lambda i: (i, 0))
    spec128 = pl.BlockSpec((RB, 2 * D), lambda i: (i, 0))
    out = pl.pallas_call(
        _tc_concat_body,
        grid=(B // RB,),
        in_specs=[spec128, spec64, spec128, spec64],
        out_specs=pl.BlockSpec((RB, 4 * D), lambda i: (i, 0)),
        out_shape=jax.ShapeDtypeStruct((B, 4 * D), jnp.float32),
    )(ri2, concat_vecs_0, ru2, concat_vecs_3)
    return out
